# initial kernel scaffold (unmeasured)
import jax
import jax.numpy as jnp
from jax import lax
from jax.experimental import pallas as pl
from jax.experimental.pallas import tpu as pltpu

N_DEV = 4
M, K, N = 4096, 4096, 2048
M_CH = M // N_DEV


def kernel(x, w_mat, scale_x, scale_w):
    def body(x_ref, w_ref, sx_ref, sw_ref, out_ref,
             acc_ref, rs_buf,
             rs_send_sems, rs_recv_sems, ag_send_sems, ag_recv_sems):
        my = lax.axis_index("i")
        left = lax.rem(my + N_DEV - 1, N_DEV)
        right = lax.rem(my + 1, N_DEV)

        for mi in range(N_DEV):
            sl = pl.ds(mi * M_CH, M_CH)
            acc_ref[sl, :] = jnp.dot(
                x_ref[sl, :], w_ref[:, :],
                preferred_element_type=jnp.float32,
            ).astype(jnp.bfloat16)

        barrier_sem = pltpu.get_barrier_semaphore()
        for nbr in (left, right):
            pl.semaphore_signal(
                barrier_sem, inc=1,
                device_id=(nbr,), device_id_type=pl.DeviceIdType.MESH,
            )
        pl.semaphore_wait(barrier_sem, 2)

        for s in range(N_DEV - 1):
            c_send = lax.rem(my + 2 * N_DEV - s, N_DEV)
            rdma = pltpu.make_async_remote_copy(
                src_ref=acc_ref.at[pl.ds(c_send * M_CH, M_CH), :],
                dst_ref=rs_buf.at[s],
                send_sem=rs_send_sems.at[s],
                recv_sem=rs_recv_sems.at[s],
                device_id=(right,),
                device_id_type=pl.DeviceIdType.MESH,
            )
            rdma.start()
            rdma.wait()
            c_recv = lax.rem(my + 2 * N_DEV - s - 1, N_DEV)
            sl = pl.ds(c_recv * M_CH, M_CH)
            acc_ref[sl, :] = acc_ref[sl, :] + rs_buf[s]

        for t in range(N_DEV - 1):
            c = lax.rem(my + 1 + 2 * N_DEV - t, N_DEV)
            sl = pl.ds(c * M_CH, M_CH)
            rdma = pltpu.make_async_remote_copy(
                src_ref=acc_ref.at[sl, :],
                dst_ref=acc_ref.at[sl, :],
                send_sem=ag_send_sems.at[t],
                recv_sem=ag_recv_sems.at[t],
                device_id=(right,),
                device_id_type=pl.DeviceIdType.MESH,
            )
            rdma.start()
            rdma.wait()

        scale = sx_ref[0] * sw_ref[0]
        for mi in range(N_DEV):
            sl = pl.ds(mi * M_CH, M_CH)
            y = acc_ref[sl, :].astype(jnp.float32) * scale
            out_ref[sl, :] = y * jax.nn.sigmoid(y)

    return pl.pallas_call(
        body,
        out_shape=jax.ShapeDtypeStruct((M, N), jnp.float32),
        in_specs=[
            pl.BlockSpec(memory_space=pltpu.VMEM),
            pl.BlockSpec(memory_space=pltpu.VMEM),
            pl.BlockSpec(memory_space=pltpu.SMEM),
            pl.BlockSpec(memory_space=pltpu.SMEM),
        ],
        out_specs=pl.BlockSpec(memory_space=pltpu.VMEM),
        scratch_shapes=[
            pltpu.VMEM((M, N), jnp.bfloat16),
            pltpu.VMEM((N_DEV - 1, M_CH, N), jnp.bfloat16),
            pltpu.SemaphoreType.DMA((N_DEV - 1,)),
            pltpu.SemaphoreType.DMA((N_DEV - 1,)),
            pltpu.SemaphoreType.DMA((N_DEV - 1,)),
            pltpu.SemaphoreType.DMA((N_DEV - 1,)),
        ],
        compiler_params=pltpu.CompilerParams(collective_id=0),
    )(x, w_mat, scale_x, scale_w)


# baseline (device time: 362357 ns/iter reference)
import jax
import jax.numpy as jnp
from jax import lax
from jax.experimental import pallas as pl
from jax.experimental.pallas import tpu as pltpu

N_DEV = 4
M, N = 4096, 2048
K_SH = 1024
M_CH = M // N_DEV
K_CH = K_SH // N_DEV


def kernel(x, w_mat, scale_x, scale_w):
    def body(x_hbm, w_hbm, sx_ref, sw_ref, out_hbm,
             acc_ref, rs_buf, x_vmem, w_dma, w_bf16, out_stage,
             local_sems,
             rs_send_sems, rs_recv_sems, ag_send_sems, ag_recv_sems):
        my = lax.axis_index("i")
        left = lax.rem(my + N_DEV - 1, N_DEV)
        right = lax.rem(my + 1, N_DEV)

        for ki in range(N_DEV):
            sl = pl.ds(ki * K_CH, K_CH)
            cp = pltpu.make_async_copy(w_hbm.at[sl, :], w_dma, local_sems.at[1])
            cp.start()
            cp.wait()
            w_bf16[sl, :] = w_dma[:, :].astype(jnp.bfloat16)

        for mi in range(N_DEV):
            sl = pl.ds(mi * M_CH, M_CH)
            cp = pltpu.make_async_copy(x_hbm.at[sl, :], x_vmem, local_sems.at[0])
            cp.start()
            cp.wait()
            acc_ref[sl, :] = jnp.dot(
                x_vmem[:, :].astype(jnp.bfloat16), w_bf16[:, :],
                preferred_element_type=jnp.float32,
            ).astype(jnp.bfloat16)

        barrier_sem = pltpu.get_barrier_semaphore()
        for nbr in (left, right):
            pl.semaphore_signal(
                barrier_sem, inc=1,
                device_id=(nbr,), device_id_type=pl.DeviceIdType.MESH,
            )
        pl.semaphore_wait(barrier_sem, 2)

        for s in range(N_DEV - 1):
            c_send = lax.rem(my + 2 * N_DEV - s, N_DEV)
            rdma = pltpu.make_async_remote_copy(
                src_ref=acc_ref.at[pl.ds(c_send * M_CH, M_CH), :],
                dst_ref=rs_buf.at[s],
                send_sem=rs_send_sems.at[s],
                recv_sem=rs_recv_sems.at[s],
                device_id=(right,),
                device_id_type=pl.DeviceIdType.MESH,
            )
            rdma.start()
            rdma.wait()
            c_recv = lax.rem(my + 2 * N_DEV - s - 1, N_DEV)
            sl = pl.ds(c_recv * M_CH, M_CH)
            acc_ref[sl, :] = acc_ref[sl, :] + rs_buf[s]

        for t in range(N_DEV - 1):
            c = lax.rem(my + 1 + 2 * N_DEV - t, N_DEV)
            sl = pl.ds(c * M_CH, M_CH)
            rdma = pltpu.make_async_remote_copy(
                src_ref=acc_ref.at[sl, :],
                dst_ref=acc_ref.at[sl, :],
                send_sem=ag_send_sems.at[t],
                recv_sem=ag_recv_sems.at[t],
                device_id=(right,),
                device_id_type=pl.DeviceIdType.MESH,
            )
            rdma.start()
            rdma.wait()

        scale = sx_ref[0] * sw_ref[0]
        for mi in range(N_DEV):
            sl = pl.ds(mi * M_CH, M_CH)
            y = acc_ref[sl, :].astype(jnp.float32) * scale
            out_stage[:, :] = y * jax.nn.sigmoid(y)
            cp = pltpu.make_async_copy(out_stage, out_hbm.at[sl, :],
                                       local_sems.at[2])
            cp.start()
            cp.wait()

    return pl.pallas_call(
        body,
        out_shape=jax.ShapeDtypeStruct((M, N), jnp.float32),
        in_specs=[
            pl.BlockSpec(memory_space=pl.ANY),
            pl.BlockSpec(memory_space=pl.ANY),
            pl.BlockSpec(memory_space=pltpu.SMEM),
            pl.BlockSpec(memory_space=pltpu.SMEM),
        ],
        out_specs=pl.BlockSpec(memory_space=pl.ANY),
        scratch_shapes=[
            pltpu.VMEM((M, N), jnp.bfloat16),
            pltpu.VMEM((N_DEV - 1, M_CH, N), jnp.bfloat16),
            pltpu.VMEM((M_CH, K_SH), jnp.float32),
            pltpu.VMEM((K_CH, N), jnp.float32),
            pltpu.VMEM((K_SH, N), jnp.bfloat16),
            pltpu.VMEM((M_CH, N), jnp.float32),
            pltpu.SemaphoreType.DMA((3,)),
            pltpu.SemaphoreType.DMA((N_DEV - 1,)),
            pltpu.SemaphoreType.DMA((N_DEV - 1,)),
            pltpu.SemaphoreType.DMA((N_DEV - 1,)),
            pltpu.SemaphoreType.DMA((N_DEV - 1,)),
        ],
        compiler_params=pltpu.CompilerParams(
            collective_id=0, vmem_limit_bytes=63 * 1024 * 1024,
        ),
    )(x, w_mat, scale_x, scale_w)


# device time: 191943 ns/iter; 1.8878x vs baseline; 1.8878x over previous
import jax
import jax.numpy as jnp
from jax import lax
from jax.experimental import pallas as pl
from jax.experimental.pallas import tpu as pltpu

N_DEV = 4
M, N = 4096, 2048
K_SH = 1024
M_CH = M // N_DEV
N_H = N // 2
K_CH = K_SH // N_DEV

L = pl.ds(0, N_H)
R = pl.ds(N_H, N_H)


def kernel(x, w_mat, scale_x, scale_w):
    def body(x_hbm, w_hbm, sx_ref, sw_ref, out_hbm,
             acc_ref, rsR_buf, rsL_buf, x_vmem, w_dma, w_bf16, out_stage,
             local_sems, rs_send, rs_recv, ag_send, ag_recv):
        my = lax.axis_index("i")
        left = lax.rem(my + N_DEV - 1, N_DEV)
        right = lax.rem(my + 1, N_DEV)

        def rows(c):
            return pl.ds(lax.rem(c + 4 * N_DEV, N_DEV) * M_CH, M_CH)

        def gemm_chunk(c):
            sl = rows(c)
            cp = pltpu.make_async_copy(x_hbm.at[sl, :], x_vmem,
                                       local_sems.at[0])
            cp.start()
            cp.wait()
            acc_ref[sl, :] = jnp.dot(
                x_vmem[:, :].astype(jnp.bfloat16), w_bf16[:, :],
                preferred_element_type=jnp.float32,
            ).astype(jnp.bfloat16)

        for ki in range(N_DEV):
            sl = pl.ds(ki * K_CH, K_CH)
            cp = pltpu.make_async_copy(w_hbm.at[sl, :], w_dma, local_sems.at[1])
            cp.start()
            cp.wait()
            w_bf16[sl, :] = w_dma[:, :].astype(jnp.bfloat16)

        gemm_chunk(my)

        barrier_sem = pltpu.get_barrier_semaphore()
        for nbr in (left, right):
            pl.semaphore_signal(
                barrier_sem, inc=1,
                device_id=(nbr,), device_id_type=pl.DeviceIdType.MESH,
            )
        pl.semaphore_wait(barrier_sem, 2)

        for s in range(N_DEV - 1):
            r_rdma = pltpu.make_async_remote_copy(
                src_ref=acc_ref.at[rows(my - s), L],
                dst_ref=rsR_buf.at[s],
                send_sem=rs_send.at[s, 0], recv_sem=rs_recv.at[s, 0],
                device_id=(right,), device_id_type=pl.DeviceIdType.MESH,
            )
            l_rdma = pltpu.make_async_remote_copy(
                src_ref=acc_ref.at[rows(my + s), R],
                dst_ref=rsL_buf.at[s],
                send_sem=rs_send.at[s, 1], recv_sem=rs_recv.at[s, 1],
                device_id=(left,), device_id_type=pl.DeviceIdType.MESH,
            )
            r_rdma.start()
            l_rdma.start()
            if s == 0:
                for o in (-1, 1, 2):
                    gemm_chunk(my + o)
            r_rdma.wait()
            l_rdma.wait()
            sl = rows(my - s - 1)
            acc_ref[sl, L] = acc_ref[sl, L] + rsR_buf[s]
            sl = rows(my + s + 1)
            acc_ref[sl, R] = acc_ref[sl, R] + rsL_buf[s]

        scale = sx_ref[0] * sw_ref[0]

        def epilogue(c, half):
            sl = rows(c)
            cols = L if half == 0 else R
            y = acc_ref[sl, cols].astype(jnp.float32) * scale
            out_stage[half, :, :] = y * jax.nn.sigmoid(y)
            cp = pltpu.make_async_copy(out_stage.at[half],
                                       out_hbm.at[sl, cols],
                                       local_sems.at[2 + half])
            cp.start()
            cp.wait()

        for t in range(N_DEV - 1):
            slR = rows(my + 1 - t)
            r_rdma = pltpu.make_async_remote_copy(
                src_ref=acc_ref.at[slR, L], dst_ref=acc_ref.at[slR, L],
                send_sem=ag_send.at[t, 0], recv_sem=ag_recv.at[t, 0],
                device_id=(right,), device_id_type=pl.DeviceIdType.MESH,
            )
            slL = rows(my - 1 + t)
            l_rdma = pltpu.make_async_remote_copy(
                src_ref=acc_ref.at[slL, R], dst_ref=acc_ref.at[slL, R],
                send_sem=ag_send.at[t, 1], recv_sem=ag_recv.at[t, 1],
                device_id=(left,), device_id_type=pl.DeviceIdType.MESH,
            )
            r_rdma.start()
            l_rdma.start()
            if t == 0:
                epilogue(my + 1, 0)
                epilogue(my - 1, 1)
            elif t == 1:
                epilogue(my, 0)
                epilogue(my, 1)
            else:
                epilogue(my - 1, 0)
                epilogue(my + 1, 1)
            r_rdma.wait()
            l_rdma.wait()
        epilogue(my + 2, 0)
        epilogue(my + 2, 1)

    return pl.pallas_call(
        body,
        out_shape=jax.ShapeDtypeStruct((M, N), jnp.float32),
        in_specs=[
            pl.BlockSpec(memory_space=pl.ANY),
            pl.BlockSpec(memory_space=pl.ANY),
            pl.BlockSpec(memory_space=pltpu.SMEM),
            pl.BlockSpec(memory_space=pltpu.SMEM),
        ],
        out_specs=pl.BlockSpec(memory_space=pl.ANY),
        scratch_shapes=[
            pltpu.VMEM((M, N), jnp.bfloat16),
            pltpu.VMEM((N_DEV - 1, M_CH, N_H), jnp.bfloat16),
            pltpu.VMEM((N_DEV - 1, M_CH, N_H), jnp.bfloat16),
            pltpu.VMEM((M_CH, K_SH), jnp.float32),
            pltpu.VMEM((K_CH, N), jnp.float32),
            pltpu.VMEM((K_SH, N), jnp.bfloat16),
            pltpu.VMEM((2, M_CH, N_H), jnp.float32),
            pltpu.SemaphoreType.DMA((4,)),
            pltpu.SemaphoreType.DMA((N_DEV - 1, 2)),
            pltpu.SemaphoreType.DMA((N_DEV - 1, 2)),
            pltpu.SemaphoreType.DMA((N_DEV - 1, 2)),
            pltpu.SemaphoreType.DMA((N_DEV - 1, 2)),
        ],
        compiler_params=pltpu.CompilerParams(
            collective_id=0, vmem_limit_bytes=63 * 1024 * 1024,
        ),
    )(x, w_mat, scale_x, scale_w)


# device time: 177371 ns/iter; 2.0429x vs baseline; 1.0822x over previous
import jax
import jax.numpy as jnp
from jax import lax
from jax.experimental import pallas as pl
from jax.experimental.pallas import tpu as pltpu

N_DEV = 4
M, N = 4096, 2048
K_SH = 1024
M_CH = M // N_DEV
M_SUB = M_CH // 2
N_H = N // 2
K_CH = K_SH // N_DEV

L = pl.ds(0, N_H)
R = pl.ds(N_H, N_H)
MESH = pl.DeviceIdType.MESH


def kernel(x, w_mat, scale_x, scale_w):
    def body(x_hbm, w_hbm, sx_ref, sw_ref, out_hbm,
             acc_ref, rsR_buf, rsL_buf, x_vmem, w_dma, w_bf16, out_stage,
             x_sems, w_sems, out_sems, rs_send, rs_recv, ag_send, ag_recv):
        my = lax.axis_index("i")
        left = lax.rem(my + N_DEV - 1, N_DEV)
        right = lax.rem(my + 1, N_DEV)

        barrier_sem = pltpu.get_barrier_semaphore()
        for nbr in (left, right):
            pl.semaphore_signal(
                barrier_sem, inc=1, device_id=(nbr,), device_id_type=MESH,
            )

        def crows(c):
            return lax.rem(c + 4 * N_DEV, N_DEV) * M_CH

        def rows(c):
            return pl.ds(crows(c), M_CH)

        def rows_sub(c, k):
            return pl.ds(crows(c) + k * M_SUB, M_SUB)

        def x_load(c, slot):
            cp = pltpu.make_async_copy(
                x_hbm.at[rows(c), :], x_vmem.at[slot], x_sems.at[slot])
            cp.start()
            return cp

        cpx = [x_load(my, 0), x_load(my - 1, 1)]
        cpw = pltpu.make_async_copy(
            w_hbm.at[pl.ds(0, K_CH), :], w_dma.at[0], w_sems.at[0])
        cpw.start()
        for ki in range(N_DEV):
            cur = ki % 2
            cpw.wait()
            if ki < N_DEV - 1:
                cpw = pltpu.make_async_copy(
                    w_hbm.at[pl.ds((ki + 1) * K_CH, K_CH), :],
                    w_dma.at[1 - cur], w_sems.at[1 - cur])
                cpw.start()
            w_bf16[pl.ds(ki * K_CH, K_CH), :] = \
                w_dma[cur, :, :].astype(jnp.bfloat16)

        def gemm_rows(dst_sl, src):
            acc_ref[dst_sl, :] = jnp.dot(
                src.astype(jnp.bfloat16), w_bf16[:, :],
                preferred_element_type=jnp.float32,
            ).astype(jnp.bfloat16)

        def rs_rdma(s, d, k):
            c = my - s if d == 0 else my + s
            buf = rsR_buf if d == 0 else rsL_buf
            return pltpu.make_async_remote_copy(
                src_ref=acc_ref.at[rows_sub(c, k), L if d == 0 else R],
                dst_ref=buf.at[s, pl.ds(k * M_SUB, M_SUB), :],
                send_sem=rs_send.at[s, d, k], recv_sem=rs_recv.at[s, d, k],
                device_id=(right if d == 0 else left,), device_id_type=MESH,
            )

        def ag_rdma(t, d, k):
            c = my + 1 - t if d == 0 else my - 1 + t
            sl_cols = (rows_sub(c, k), L if d == 0 else R)
            return pltpu.make_async_remote_copy(
                src_ref=acc_ref.at[sl_cols[0], sl_cols[1]],
                dst_ref=acc_ref.at[sl_cols[0], sl_cols[1]],
                send_sem=ag_send.at[t, d, k], recv_sem=ag_recv.at[t, d, k],
                device_id=(right if d == 0 else left,), device_id_type=MESH,
            )

        cpx[0].wait()
        gemm_rows(rows_sub(my, 0), x_vmem[0, pl.ds(0, M_SUB), :])
        pl.semaphore_wait(barrier_sem, 2)
        pend_rs = {}
        for d in (0, 1):
            pend_rs[(0, d, 0)] = rs_rdma(0, d, 0)
            pend_rs[(0, d, 0)].start()
        gemm_rows(rows_sub(my, 1), x_vmem[0, pl.ds(M_SUB, M_SUB), :])
        for d in (0, 1):
            pend_rs[(0, d, 1)] = rs_rdma(0, d, 1)
            pend_rs[(0, d, 1)].start()

        next_c = [my + 1, my + 2]
        for idx, o in enumerate((-1, 1, 2)):
            slot = (idx + 1) % 2
            cpx[slot].wait()
            if idx < 2:
                cpx[1 - slot] = x_load(next_c[idx], 1 - slot)
            gemm_rows(rows(my + o), x_vmem[slot, :, :])

        pend_ag = {}
        for s in range(N_DEV - 1):
            for k in (0, 1):
                for d in (0, 1):
                    pend_rs[(s, d, k)].wait()
                    c_r = my - s - 1 if d == 0 else my + s + 1
                    buf = rsR_buf if d == 0 else rsL_buf
                    sl = rows_sub(c_r, k)
                    cols = L if d == 0 else R
                    acc_ref[sl, cols] = (
                        acc_ref[sl, cols]
                        + buf[s, pl.ds(k * M_SUB, M_SUB), :])
                    if s < N_DEV - 2:
                        nxt = rs_rdma(s + 1, d, k)
                        nxt.start()
                        pend_rs[(s + 1, d, k)] = nxt
                    else:
                        ag0 = ag_rdma(0, d, k)
                        ag0.start()
                        pend_ag[(0, d, k)] = ag0

        scale = sx_ref[0] * sw_ref[0]

        def epi_sub(c, half, k):
            sl = rows_sub(c, k)
            cols = L if half == 0 else R
            y = acc_ref[sl, cols].astype(jnp.float32) * scale
            out_stage[half, :, :] = y * jax.nn.sigmoid(y)
            cp = pltpu.make_async_copy(
                out_stage.at[half], out_hbm.at[sl, cols], out_sems.at[half])
            cp.start()
            cp.wait()

        def epi(c, half):
            epi_sub(c, half, 0)
            epi_sub(c, half, 1)

        epi(my + 1, 0)
        epi(my - 1, 1)
        for t in (0, 1):
            for k in (0, 1):
                for d in (0, 1):
                    pend_ag[(t, d, k)].wait()
                    nxt = ag_rdma(t + 1, d, k)
                    nxt.start()
                    pend_ag[(t + 1, d, k)] = nxt
            if t == 0:
                epi(my, 0)
                epi(my, 1)
            else:
                epi(my - 1, 0)
                epi(my + 1, 1)
        for k in (0, 1):
            for d in (0, 1):
                pend_ag[(2, d, k)].wait()
            epi_sub(my + 2, 0, k)
            epi_sub(my + 2, 1, k)

    return pl.pallas_call(
        body,
        out_shape=jax.ShapeDtypeStruct((M, N), jnp.float32),
        in_specs=[
            pl.BlockSpec(memory_space=pl.ANY),
            pl.BlockSpec(memory_space=pl.ANY),
            pl.BlockSpec(memory_space=pltpu.SMEM),
            pl.BlockSpec(memory_space=pltpu.SMEM),
        ],
        out_specs=pl.BlockSpec(memory_space=pl.ANY),
        scratch_shapes=[
            pltpu.VMEM((M, N), jnp.bfloat16),
            pltpu.VMEM((N_DEV - 1, M_CH, N_H), jnp.bfloat16),
            pltpu.VMEM((N_DEV - 1, M_CH, N_H), jnp.bfloat16),
            pltpu.VMEM((2, M_CH, K_SH), jnp.float32),
            pltpu.VMEM((2, K_CH, N), jnp.float32),
            pltpu.VMEM((K_SH, N), jnp.bfloat16),
            pltpu.VMEM((2, M_SUB, N_H), jnp.float32),
            pltpu.SemaphoreType.DMA((2,)),
            pltpu.SemaphoreType.DMA((2,)),
            pltpu.SemaphoreType.DMA((2,)),
            pltpu.SemaphoreType.DMA((N_DEV - 1, 2, 2)),
            pltpu.SemaphoreType.DMA((N_DEV - 1, 2, 2)),
            pltpu.SemaphoreType.DMA((N_DEV - 1, 2, 2)),
            pltpu.SemaphoreType.DMA((N_DEV - 1, 2, 2)),
        ],
        compiler_params=pltpu.CompilerParams(
            collective_id=0, vmem_limit_bytes=63 * 1024 * 1024,
        ),
    )(x, w_mat, scale_x, scale_w)


# device time: 171022 ns/iter; 2.1188x vs baseline; 1.0371x over previous
import jax
import jax.numpy as jnp
from jax import lax
from jax.experimental import pallas as pl
from jax.experimental.pallas import tpu as pltpu

N_DEV = 4
M, N = 4096, 2048
K_SH = 1024
M_CH = M // N_DEV
M_SUB = M_CH // 2
N_H = N // 2
K_CH = K_SH // N_DEV

L = pl.ds(0, N_H)
R = pl.ds(N_H, N_H)
MESH = pl.DeviceIdType.MESH


def kernel(x, w_mat, scale_x, scale_w):
    def body(x_hbm, w_hbm, sx_ref, sw_ref, out_hbm,
             acc_ref, rsR_buf, rsL_buf, x_vmem, w_dma, w_bf16, out_stage,
             x_sems, w_sems, out_sems, rs_send, rs_recv, ag_send, ag_recv):
        my = lax.axis_index("i")
        left = lax.rem(my + N_DEV - 1, N_DEV)
        right = lax.rem(my + 1, N_DEV)

        barrier_sem = pltpu.get_barrier_semaphore()
        for nbr in (left, right):
            pl.semaphore_signal(
                barrier_sem, inc=1, device_id=(nbr,), device_id_type=MESH,
            )

        def crows(c):
            return lax.rem(c + 4 * N_DEV, N_DEV) * M_CH

        def rows(c):
            return pl.ds(crows(c), M_CH)

        def rows_sub(c, k):
            return pl.ds(crows(c) + k * M_SUB, M_SUB)

        def x_load(c, slot):
            cp = pltpu.make_async_copy(
                x_hbm.at[rows(c), :], x_vmem.at[slot], x_sems.at[slot])
            cp.start()
            return cp

        cpx = [x_load(my, 0), x_load(my - 1, 1)]
        cpw = pltpu.make_async_copy(
            w_hbm.at[pl.ds(0, K_CH), :], w_dma.at[0], w_sems.at[0])
        cpw.start()
        for ki in range(N_DEV):
            cur = ki % 2
            cpw.wait()
            if ki < N_DEV - 1:
                cpw = pltpu.make_async_copy(
                    w_hbm.at[pl.ds((ki + 1) * K_CH, K_CH), :],
                    w_dma.at[1 - cur], w_sems.at[1 - cur])
                cpw.start()
            w_bf16[pl.ds(ki * K_CH, K_CH), :] = \
                w_dma[cur, :, :].astype(jnp.bfloat16)

        def gemm_rows(dst_sl, src):
            acc_ref[dst_sl, :] = jnp.dot(
                src.astype(jnp.bfloat16), w_bf16[:, :],
                preferred_element_type=jnp.float32,
            ).astype(jnp.bfloat16)

        def rs_rdma(s, d, k):
            c = my - s if d == 0 else my + s
            buf = rsR_buf if d == 0 else rsL_buf
            return pltpu.make_async_remote_copy(
                src_ref=acc_ref.at[rows_sub(c, k), L if d == 0 else R],
                dst_ref=buf.at[s, pl.ds(k * M_SUB, M_SUB), :],
                send_sem=rs_send.at[s, d, k], recv_sem=rs_recv.at[s, d, k],
                device_id=(right if d == 0 else left,), device_id_type=MESH,
            )

        def ag_rdma(t, d, k):
            c = my + 1 - t if d == 0 else my - 1 + t
            sl_cols = (rows_sub(c, k), L if d == 0 else R)
            return pltpu.make_async_remote_copy(
                src_ref=acc_ref.at[sl_cols[0], sl_cols[1]],
                dst_ref=acc_ref.at[sl_cols[0], sl_cols[1]],
                send_sem=ag_send.at[t, d, k], recv_sem=ag_recv.at[t, d, k],
                device_id=(right if d == 0 else left,), device_id_type=MESH,
            )

        cpx[0].wait()
        gemm_rows(rows_sub(my, 0), x_vmem[0, pl.ds(0, M_SUB), :])
        pl.semaphore_wait(barrier_sem, 2)
        pend_rs = {}
        for d in (0, 1):
            pend_rs[(0, d, 0)] = rs_rdma(0, d, 0)
            pend_rs[(0, d, 0)].start()
        gemm_rows(rows_sub(my, 1), x_vmem[0, pl.ds(M_SUB, M_SUB), :])
        for d in (0, 1):
            pend_rs[(0, d, 1)] = rs_rdma(0, d, 1)
            pend_rs[(0, d, 1)].start()

        def gemm_under_hop(idx, o, slot):
            cpx[slot].wait()
            if idx < 2:
                cpx[1 - slot] = x_load([my + 1, my + 2][idx], 1 - slot)
            gemm_rows(rows(my + o), x_vmem[slot, :, :])

        gemm_under_hop(0, -1, 1)
        gemm_under_hop(1, 1, 0)

        pend_ag = {}
        for s in range(N_DEV - 1):
            for k in (0, 1):
                if s == 1 and k == 0:
                    gemm_under_hop(2, 2, 1)
                for d in (0, 1):
                    pend_rs[(s, d, k)].wait()
                    c_r = my - s - 1 if d == 0 else my + s + 1
                    buf = rsR_buf if d == 0 else rsL_buf
                    sl = rows_sub(c_r, k)
                    cols = L if d == 0 else R
                    acc_ref[sl, cols] = (
                        acc_ref[sl, cols]
                        + buf[s, pl.ds(k * M_SUB, M_SUB), :])
                    if s < N_DEV - 2:
                        nxt = rs_rdma(s + 1, d, k)
                        nxt.start()
                        pend_rs[(s + 1, d, k)] = nxt
                    else:
                        ag0 = ag_rdma(0, d, k)
                        ag0.start()
                        pend_ag[(0, d, k)] = ag0

        scale = sx_ref[0] * sw_ref[0]

        pend_out = [None, None]

        def epi_sub(c, half, k):
            sl = rows_sub(c, k)
            cols = L if half == 0 else R
            if pend_out[half] is not None:
                pend_out[half].wait()
            y = acc_ref[sl, cols].astype(jnp.float32) * scale
            out_stage[half, :, :] = (y * jax.nn.sigmoid(y)).astype(jnp.bfloat16)
            cp = pltpu.make_async_copy(
                out_stage.at[half], out_hbm.at[sl, cols], out_sems.at[half])
            cp.start()
            pend_out[half] = cp

        def epi(c, half):
            epi_sub(c, half, 0)
            epi_sub(c, half, 1)

        epi(my + 1, 0)
        epi(my - 1, 1)
        for t in (0, 1):
            for k in (0, 1):
                for d in (0, 1):
                    pend_ag[(t, d, k)].wait()
                    nxt = ag_rdma(t + 1, d, k)
                    nxt.start()
                    pend_ag[(t + 1, d, k)] = nxt
            if t == 0:
                epi(my, 0)
                epi(my, 1)
            else:
                epi(my - 1, 0)
                epi(my + 1, 1)
        for k in (0, 1):
            for d in (0, 1):
                pend_ag[(2, d, k)].wait()
            epi_sub(my + 2, 0, k)
            epi_sub(my + 2, 1, k)
        pend_out[0].wait()
        pend_out[1].wait()

    out = pl.pallas_call(
        body,
        out_shape=jax.ShapeDtypeStruct((M, N), jnp.bfloat16),
        in_specs=[
            pl.BlockSpec(memory_space=pl.ANY),
            pl.BlockSpec(memory_space=pl.ANY),
            pl.BlockSpec(memory_space=pltpu.SMEM),
            pl.BlockSpec(memory_space=pltpu.SMEM),
        ],
        out_specs=pl.BlockSpec(memory_space=pl.ANY),
        scratch_shapes=[
            pltpu.VMEM((M, N), jnp.bfloat16),
            pltpu.VMEM((N_DEV - 1, M_CH, N_H), jnp.bfloat16),
            pltpu.VMEM((N_DEV - 1, M_CH, N_H), jnp.bfloat16),
            pltpu.VMEM((2, M_CH, K_SH), jnp.float32),
            pltpu.VMEM((2, K_CH, N), jnp.float32),
            pltpu.VMEM((K_SH, N), jnp.bfloat16),
            pltpu.VMEM((2, M_SUB, N_H), jnp.bfloat16),
            pltpu.SemaphoreType.DMA((2,)),
            pltpu.SemaphoreType.DMA((2,)),
            pltpu.SemaphoreType.DMA((2,)),
            pltpu.SemaphoreType.DMA((N_DEV - 1, 2, 2)),
            pltpu.SemaphoreType.DMA((N_DEV - 1, 2, 2)),
            pltpu.SemaphoreType.DMA((N_DEV - 1, 2, 2)),
            pltpu.SemaphoreType.DMA((N_DEV - 1, 2, 2)),
        ],
        compiler_params=pltpu.CompilerParams(
            collective_id=0, vmem_limit_bytes=63 * 1024 * 1024,
        ),
    )(x, w_mat, scale_x, scale_w)
    return out.astype(jnp.float32)


# device time: 169348 ns/iter; 2.1397x vs baseline; 1.0099x over previous
import jax
import jax.numpy as jnp
from jax import lax
from jax.experimental import pallas as pl
from jax.experimental.pallas import tpu as pltpu

N_DEV = 4
M, N = 4096, 2048
K_SH = 1024
M_CH = M // N_DEV
M_SUB = M_CH // 2
N_H = N // 2
K_CH = K_SH // N_DEV

L = pl.ds(0, N_H)
R = pl.ds(N_H, N_H)
MESH = pl.DeviceIdType.MESH


def kernel(x, w_mat, scale_x, scale_w):
    def body(x_hbm, w_hbm, sx_ref, sw_ref, out_hbm,
             acc_ref, rsR_buf, rsL_buf, x_vmem, w_dma, w_bf16, out_stage,
             acc_f32, x_sems, w_sems, out_sems,
             rs_send, rs_recv, ag_send, ag_recv):
        my = lax.axis_index("i")
        left = lax.rem(my + N_DEV - 1, N_DEV)
        right = lax.rem(my + 1, N_DEV)

        barrier_sem = pltpu.get_barrier_semaphore()
        for nbr in (left, right):
            pl.semaphore_signal(
                barrier_sem, inc=1, device_id=(nbr,), device_id_type=MESH,
            )

        def crows(c):
            return lax.rem(c + 4 * N_DEV, N_DEV) * M_CH

        def rows(c):
            return pl.ds(crows(c), M_CH)

        def rows_sub(c, k):
            return pl.ds(crows(c) + k * M_SUB, M_SUB)

        def x_load(c, slot):
            cp = pltpu.make_async_copy(
                x_hbm.at[rows(c), :], x_vmem.at[slot], x_sems.at[slot])
            cp.start()
            return cp

        cpx = [x_load(my, 0), x_load(my - 1, 1)]
        cpw = pltpu.make_async_copy(
            w_hbm.at[pl.ds(0, K_CH), :], w_dma.at[0], w_sems.at[0])
        cpw.start()
        cpx[0].wait()
        for ki in range(N_DEV):
            cur = ki % 2
            cpw.wait()
            if ki < N_DEV - 1:
                cpw = pltpu.make_async_copy(
                    w_hbm.at[pl.ds((ki + 1) * K_CH, K_CH), :],
                    w_dma.at[1 - cur], w_sems.at[1 - cur])
                cpw.start()
            w_bf16[pl.ds(ki * K_CH, K_CH), :] = \
                w_dma[cur, :, :].astype(jnp.bfloat16)
            part = jnp.dot(
                x_vmem[0, pl.ds(0, M_SUB), pl.ds(ki * K_CH, K_CH)
                       ].astype(jnp.bfloat16),
                w_bf16[pl.ds(ki * K_CH, K_CH), :],
                preferred_element_type=jnp.float32)
            acc_f32[:, :] = part if ki == 0 else acc_f32[:, :] + part

        def gemm_rows(dst_sl, src):
            acc_ref[dst_sl, :] = jnp.dot(
                src.astype(jnp.bfloat16), w_bf16[:, :],
                preferred_element_type=jnp.float32,
            ).astype(jnp.bfloat16)

        def rs_rdma(s, d, k):
            c = my - s if d == 0 else my + s
            buf = rsR_buf if d == 0 else rsL_buf
            return pltpu.make_async_remote_copy(
                src_ref=acc_ref.at[rows_sub(c, k), L if d == 0 else R],
                dst_ref=buf.at[s, pl.ds(k * M_SUB, M_SUB), :],
                send_sem=rs_send.at[s, d, k], recv_sem=rs_recv.at[s, d, k],
                device_id=(right if d == 0 else left,), device_id_type=MESH,
            )

        def ag_rdma(t, d, k):
            c = my + 1 - t if d == 0 else my - 1 + t
            sl_cols = (rows_sub(c, k), L if d == 0 else R)
            return pltpu.make_async_remote_copy(
                src_ref=acc_ref.at[sl_cols[0], sl_cols[1]],
                dst_ref=acc_ref.at[sl_cols[0], sl_cols[1]],
                send_sem=ag_send.at[t, d, k], recv_sem=ag_recv.at[t, d, k],
                device_id=(right if d == 0 else left,), device_id_type=MESH,
            )

        acc_ref[rows_sub(my, 0), :] = acc_f32[:, :].astype(jnp.bfloat16)
        pl.semaphore_wait(barrier_sem, 2)
        pend_rs = {}
        for d in (0, 1):
            pend_rs[(0, d, 0)] = rs_rdma(0, d, 0)
            pend_rs[(0, d, 0)].start()
        gemm_rows(rows_sub(my, 1), x_vmem[0, pl.ds(M_SUB, M_SUB), :])
        for d in (0, 1):
            pend_rs[(0, d, 1)] = rs_rdma(0, d, 1)
            pend_rs[(0, d, 1)].start()

        def gemm_under_hop(idx, o, slot):
            cpx[slot].wait()
            if idx < 2:
                cpx[1 - slot] = x_load([my + 1, my + 2][idx], 1 - slot)
            gemm_rows(rows(my + o), x_vmem[slot, :, :])

        gemm_under_hop(0, -1, 1)
        gemm_under_hop(1, 1, 0)

        pend_ag = {}
        for s in range(N_DEV - 1):
            for k in (0, 1):
                if s == 1 and k == 0:
                    gemm_under_hop(2, 2, 1)
                for d in (0, 1):
                    pend_rs[(s, d, k)].wait_recv()
                    c_r = my - s - 1 if d == 0 else my + s + 1
                    buf = rsR_buf if d == 0 else rsL_buf
                    sl = rows_sub(c_r, k)
                    cols = L if d == 0 else R
                    acc_ref[sl, cols] = (
                        acc_ref[sl, cols]
                        + buf[s, pl.ds(k * M_SUB, M_SUB), :])
                    if s < N_DEV - 2:
                        nxt = rs_rdma(s + 1, d, k)
                        nxt.start()
                        pend_rs[(s + 1, d, k)] = nxt
                    else:
                        ag0 = ag_rdma(0, d, k)
                        ag0.start()
                        pend_ag[(0, d, k)] = ag0

        scale = sx_ref[0] * sw_ref[0]

        pend_out = [None, None]

        def epi_sub(c, half, k):
            sl = rows_sub(c, k)
            cols = L if half == 0 else R
            if pend_out[half] is not None:
                pend_out[half].wait()
            y = acc_ref[sl, cols].astype(jnp.float32) * scale
            out_stage[half, :, :] = (y * jax.nn.sigmoid(y)).astype(jnp.bfloat16)
            cp = pltpu.make_async_copy(
                out_stage.at[half], out_hbm.at[sl, cols], out_sems.at[half])
            cp.start()
            pend_out[half] = cp

        def epi(c, half):
            epi_sub(c, half, 0)
            epi_sub(c, half, 1)

        epi(my + 1, 0)
        epi(my - 1, 1)
        for t in (0, 1):
            for k in (0, 1):
                for d in (0, 1):
                    pend_ag[(t, d, k)].wait_recv()
                    nxt = ag_rdma(t + 1, d, k)
                    nxt.start()
                    pend_ag[(t + 1, d, k)] = nxt
            if t == 0:
                epi(my, 0)
                epi(my, 1)
            else:
                epi(my - 1, 0)
                epi(my + 1, 1)
        for k in (0, 1):
            for d in (0, 1):
                pend_ag[(2, d, k)].wait_recv()
            epi_sub(my + 2, 0, k)
            epi_sub(my + 2, 1, k)
        for obj in list(pend_rs.values()) + list(pend_ag.values()):
            obj.wait_send()
        pend_out[0].wait()
        pend_out[1].wait()

    out = pl.pallas_call(
        body,
        out_shape=jax.ShapeDtypeStruct((M, N), jnp.bfloat16),
        in_specs=[
            pl.BlockSpec(memory_space=pl.ANY),
            pl.BlockSpec(memory_space=pl.ANY),
            pl.BlockSpec(memory_space=pltpu.SMEM),
            pl.BlockSpec(memory_space=pltpu.SMEM),
        ],
        out_specs=pl.BlockSpec(memory_space=pl.ANY),
        scratch_shapes=[
            pltpu.VMEM((M, N), jnp.bfloat16),
            pltpu.VMEM((N_DEV - 1, M_CH, N_H), jnp.bfloat16),
            pltpu.VMEM((N_DEV - 1, M_CH, N_H), jnp.bfloat16),
            pltpu.VMEM((2, M_CH, K_SH), jnp.float32),
            pltpu.VMEM((2, K_CH, N), jnp.float32),
            pltpu.VMEM((K_SH, N), jnp.bfloat16),
            pltpu.VMEM((2, M_SUB, N_H), jnp.bfloat16),
            pltpu.VMEM((M_SUB, N), jnp.float32),
            pltpu.SemaphoreType.DMA((2,)),
            pltpu.SemaphoreType.DMA((2,)),
            pltpu.SemaphoreType.DMA((2,)),
            pltpu.SemaphoreType.DMA((N_DEV - 1, 2, 2)),
            pltpu.SemaphoreType.DMA((N_DEV - 1, 2, 2)),
            pltpu.SemaphoreType.DMA((N_DEV - 1, 2, 2)),
            pltpu.SemaphoreType.DMA((N_DEV - 1, 2, 2)),
        ],
        compiler_params=pltpu.CompilerParams(
            collective_id=0, vmem_limit_bytes=63 * 1024 * 1024,
        ),
    )(x, w_mat, scale_x, scale_w)
    return out.astype(jnp.float32)
